# baseline (speedup 1.0000x reference)
import jax
import jax.numpy as jnp
from jax import lax
from jax.experimental import pallas as pl
from jax.experimental.pallas import tpu as pltpu
from jax.experimental.pallas import tpu_sc as plsc

_DEPTH = 1000
_RC = 40
_NCH = _DEPTH // _RC


def _onehot_sc_body(x_hbm, out_hbm, idx_v, buf0, buf1, sem0, sem1):
    nc = lax.axis_size("c")
    wid = lax.axis_index("s") * nc + lax.axis_index("c")
    cols_per_w = idx_v.shape[0]
    col0 = wid * cols_per_w

    pltpu.sync_copy(x_hbm.at[pl.ds(col0, cols_per_w)], idx_v)

    iota16 = lax.broadcasted_iota(jnp.int32, (16,), 0)
    ones_v = jnp.ones((16,), jnp.float32)
    zeros_v = jnp.zeros((16,), jnp.float32)

    def _zero_row(r, _):
        for buf in (buf0, buf1):
            for j in range(cols_per_w // 16):
                buf[r, pl.ds(j * 16, 16)] = zeros_v
        return 0
    lax.fori_loop(0, _RC, _zero_row, 0)

    bufs = (buf0, buf1)
    sems = (sem0, sem1)

    def _scatter(buf, c, vals):
        row0 = c * _RC
        for k in range(cols_per_w // 16):
            x = idx_v[pl.ds(k * 16, 16)]
            rloc = x - row0
            mask = (rloc >= 0) & (rloc < _RC)
            rsafe = jnp.clip(rloc, 0, _RC - 1)
            plsc.store_scatter(buf, [rsafe, k * 16 + iota16], vals, mask=mask)

    def _dma(b, c):
        dst = out_hbm.at[pl.ds(c * _RC, _RC), pl.ds(col0, cols_per_w)]
        return pltpu.async_copy(bufs[b], dst, sems[b])

    def _wait(b, c):
        dst = out_hbm.at[pl.ds(c * _RC, _RC), pl.ds(col0, cols_per_w)]
        pltpu.make_async_copy(bufs[b], dst, sems[b]).wait()

    _scatter(buf0, 0, ones_v)
    _dma(0, 0)
    _scatter(buf1, 1, ones_v)
    _dma(1, 1)

    def _pair(t, _):
        for b in (0, 1):
            c = 2 * t + 2 + b
            _wait(b, c - 2)
            _scatter(bufs[b], c - 2, zeros_v)

            @pl.when(c < _NCH)
            def _():
                _scatter(bufs[b], c, ones_v)
                _dma(b, c)
        return 0
    lax.fori_loop(0, (_NCH - 1) // 2, _pair, 0)

    _wait(0, _NCH - 1)


def kernel(X_in, ones):
    del ones
    batch = X_in.shape[0]
    info = plsc.get_sparse_core_info()
    nw = info.num_cores * info.num_subcores
    cols_per_w = batch // nw
    mesh = plsc.VectorSubcoreMesh(core_axis_name="c", subcore_axis_name="s")
    sc_call = pl.kernel(
        _onehot_sc_body,
        out_type=jax.ShapeDtypeStruct((_DEPTH, batch), jnp.float32),
        mesh=mesh,
        scratch_types=[
            pltpu.VMEM((cols_per_w,), jnp.int32),
            pltpu.VMEM((_RC, cols_per_w), jnp.float32),
            pltpu.VMEM((_RC, cols_per_w), jnp.float32),
            pltpu.SemaphoreType.DMA,
            pltpu.SemaphoreType.DMA,
        ],
        compiler_params=pltpu.CompilerParams(
            needs_layout_passes=False, use_tc_tiling_on_sc=True,
            skip_device_barrier=True),
    )
    outT = sc_call(X_in.astype(jnp.int32))
    return outT.T


# speedup vs baseline: 1.0028x; 1.0028x over previous
import jax
import jax.numpy as jnp
from jax import lax
from jax.experimental import pallas as pl
from jax.experimental.pallas import tpu as pltpu
from jax.experimental.pallas import tpu_sc as plsc

_DEPTH = 1000
_RC = 40
_NCH = _DEPTH // _RC


def _onehot_sc_body(x_hbm, out_hbm, idx_v, buf0, buf1, sem0, sem1):
    nc = lax.axis_size("c")
    wid = lax.axis_index("s") * nc + lax.axis_index("c")
    cols_per_w = idx_v.shape[0]
    col0 = wid * cols_per_w

    pltpu.sync_copy(x_hbm.at[pl.ds(col0, cols_per_w)], idx_v)

    iota16 = lax.broadcasted_iota(jnp.int32, (16,), 0)
    ones_v = jnp.ones((16,), jnp.float32)
    zeros_v = jnp.zeros((16,), jnp.float32)

    def _zero_row(r, _):
        for buf in (buf0, buf1):
            for j in range(cols_per_w // 16):
                buf[r, pl.ds(j * 16, 16)] = zeros_v
        return 0
    lax.fori_loop(0, _RC, _zero_row, 0)

    bufs = (buf0, buf1)
    sems = (sem0, sem1)

    def _scatter(buf, c, vals):
        row0 = c * _RC
        for k in range(cols_per_w // 16):
            x = idx_v[pl.ds(k * 16, 16)]
            rloc = x - row0
            mask = (rloc >= 0) & (rloc < _RC)
            rsafe = jnp.clip(rloc, 0, _RC - 1)
            plsc.store_scatter(buf, [rsafe, k * 16 + iota16], vals, mask=mask)

    def _dma(b, c):
        dst = out_hbm.at[pl.ds(c * _RC, _RC), pl.ds(col0, cols_per_w)]
        return pltpu.async_copy(bufs[b], dst, sems[b])

    def _wait(b, c):
        dst = out_hbm.at[pl.ds(c * _RC, _RC), pl.ds(col0, cols_per_w)]
        pltpu.make_async_copy(bufs[b], dst, sems[b]).wait()

    _scatter(buf0, 0, ones_v)
    _dma(0, 0)
    _scatter(buf1, 1, ones_v)
    _dma(1, 1)

    def _pair(t, _):
        for b in (0, 1):
            c = 2 * t + 2 + b
            _wait(b, c - 2)
            _scatter(bufs[b], c - 2, zeros_v)

            @pl.when(c < _NCH)
            def _():
                _scatter(bufs[b], c, ones_v)
                _dma(b, c)
        return 0
    lax.fori_loop(0, (_NCH - 1) // 2, _pair, 0)

    _wait(0, _NCH - 1)


def kernel(X_in, ones):
    del ones
    batch = X_in.shape[0]
    info = plsc.get_sparse_core_info()
    nw = info.num_cores * info.num_subcores
    cols_per_w = batch // nw
    mesh = plsc.VectorSubcoreMesh(core_axis_name="c", subcore_axis_name="s")
    sc_call = pl.kernel(
        _onehot_sc_body,
        out_type=jax.ShapeDtypeStruct((_DEPTH, batch), jnp.float32),
        mesh=mesh,
        scratch_types=[
            pltpu.VMEM((cols_per_w,), jnp.int32),
            pltpu.VMEM((_RC, cols_per_w), jnp.float32),
            pltpu.VMEM((_RC, cols_per_w), jnp.float32),
            pltpu.SemaphoreType.DMA,
            pltpu.SemaphoreType.DMA,
        ],
        compiler_params=pltpu.CompilerParams(
            needs_layout_passes=False, use_tc_tiling_on_sc=True),
    )
    outT = sc_call(X_in.astype(jnp.int32))
    return outT.T
